# baseline (speedup 1.0000x reference)
import jax
import jax.numpy as jnp
from jax.experimental import pallas as pl
from jax.experimental.pallas import tpu as pltpu

WAY = 5
K = 3
NEG = -1e9
EPS = 1e-30
CHUNK = 64


def _support_prep_kernel(s_ref, msb_ref):
    s = s_ref[...]
    nrm = jnp.sqrt(jnp.sum(s * s, axis=1, keepdims=True))
    ns = s / jnp.maximum(nrm, EPS)
    hw = s.shape[2]
    rows = []
    for i in range(WAY):
        for j in range(WAY):
            rows.append(jnp.sum(ns[i] * ns[j], axis=0, keepdims=True))
    sim = jnp.concatenate(rows, axis=0)
    sim = sim - jnp.max(sim, axis=1, keepdims=True)
    e = jnp.exp(sim)
    w = e / jnp.sum(e, axis=1, keepdims=True)
    aggs = [jnp.mean(w[i * WAY:(i + 1) * WAY], axis=0, keepdims=True)
            for i in range(WAY)]
    agg = jnp.concatenate(aggs, axis=0)
    n = WAY * hw
    mean = jnp.sum(agg) / n
    var = jnp.sum((agg - mean) ** 2) / (n - 1)
    thr = mean - jnp.sqrt(var)
    mask = (agg > thr).astype(jnp.float32)
    msb_ref[...] = (ns * mask[:, None, :]).astype(jnp.bfloat16)


def _query_agg_kernel(x_ref, ms_ref, agg_ref, nq_ref):
    qb, c, hw = x_ref.shape
    for qi in range(qb):
        acc = None
        for cc in range(0, c, CHUNK):
            xc = x_ref[qi, cc:cc + CHUNK, :]
            p = jnp.sum(xc * xc, axis=0, keepdims=True)
            acc = p if acc is None else acc + p
        inv = 1.0 / jnp.maximum(jnp.sqrt(acc), EPS)
        sims = [None] * WAY
        for cc in range(0, c, CHUNK):
            xc = x_ref[qi, cc:cc + CHUNK, :] * inv
            nq_ref[qi, cc:cc + CHUNK, :] = xc.astype(jnp.bfloat16)
            for w in range(WAY):
                msc = ms_ref[w, cc:cc + CHUNK, :].astype(jnp.float32)
                p = jnp.sum(xc * msc, axis=0, keepdims=True)
                sims[w] = p if sims[w] is None else sims[w] + p
        sim = jnp.concatenate(sims, axis=0)
        sim = sim - jnp.max(sim, axis=1, keepdims=True)
        e = jnp.exp(sim)
        w_ = e / jnp.sum(e, axis=1, keepdims=True)
        agg_ref[qi] = jnp.mean(w_, axis=0, keepdims=True)


def _thr_kernel(agg_ref, thr_ref):
    a = agg_ref[...]
    n = a.shape[0] * a.shape[2]
    mean = jnp.sum(a) / n
    var = jnp.sum((a - mean) ** 2) / (n - 1)
    thr_ref[...] = jnp.broadcast_to(mean - jnp.sqrt(var), (1, 1))


def _top3_rowsum(inner):
    m1 = jnp.max(inner, axis=1, keepdims=True)
    ge1 = inner >= m1
    c1 = jnp.sum(ge1.astype(jnp.float32), axis=1, keepdims=True)
    x2 = jnp.where(ge1, NEG, inner)
    m2 = jnp.max(x2, axis=1, keepdims=True)
    ge2 = x2 >= m2
    c2 = jnp.sum(ge2.astype(jnp.float32), axis=1, keepdims=True)
    x3 = jnp.where(ge2, NEG, x2)
    m3 = jnp.max(x3, axis=1, keepdims=True)
    kf = float(K)
    t1 = jnp.minimum(c1, kf)
    t2 = jnp.minimum(c2, jnp.maximum(kf - t1, 0.0))
    t3 = jnp.maximum(kf - t1 - t2, 0.0)
    return m1 * t1 + m2 * t2 + m3 * t3


def _main_kernel(nq_ref, msb_ref, agg_ref, thr_ref, out_ref):
    thr = thr_ref[0, 0]
    mask = (agg_ref[0] > thr).astype(jnp.bfloat16)
    mq = nq_ref[0] * mask
    sums = []
    for w in range(WAY):
        inner = jax.lax.dot_general(
            mq, msb_ref[w], (((0,), (0,)), ((), ())),
            preferred_element_type=jnp.float32)
        sums.append(jnp.sum(_top3_rowsum(inner), axis=0, keepdims=True))
    out_ref[...] = jnp.concatenate(sums, axis=1)[None]


def kernel(x1, x2):
    q, c, h, w = x1.shape[0], x1.shape[1], x1.shape[2], x1.shape[3]
    hw = h * w
    xq = x1.reshape(q, c, hw)
    xs = x2.reshape(WAY, c, hw)

    msb = pl.pallas_call(
        _support_prep_kernel,
        out_shape=jax.ShapeDtypeStruct((WAY, c, hw), jnp.bfloat16),
    )(xs)

    qb = 8
    agg, nq = pl.pallas_call(
        _query_agg_kernel,
        grid=(q // qb,),
        in_specs=[
            pl.BlockSpec((qb, c, hw), lambda i: (i, 0, 0)),
            pl.BlockSpec((WAY, c, hw), lambda i: (0, 0, 0)),
        ],
        out_specs=(pl.BlockSpec((qb, 1, hw), lambda i: (i, 0, 0)),
                   pl.BlockSpec((qb, c, hw), lambda i: (i, 0, 0))),
        out_shape=(jax.ShapeDtypeStruct((q, 1, hw), jnp.float32),
                   jax.ShapeDtypeStruct((q, c, hw), jnp.bfloat16)),
        compiler_params=pltpu.CompilerParams(
            dimension_semantics=("arbitrary",)),
    )(xq, msb)

    thr = pl.pallas_call(
        _thr_kernel,
        out_shape=jax.ShapeDtypeStruct((1, 1), jnp.float32),
    )(agg)

    out = pl.pallas_call(
        _main_kernel,
        grid=(q,),
        in_specs=[
            pl.BlockSpec((1, c, hw), lambda i: (i, 0, 0)),
            pl.BlockSpec((WAY, c, hw), lambda i: (0, 0, 0)),
            pl.BlockSpec((1, 1, hw), lambda i: (i, 0, 0)),
            pl.BlockSpec((1, 1), lambda i: (0, 0)),
        ],
        out_specs=pl.BlockSpec((1, 1, WAY), lambda i: (i, 0, 0)),
        out_shape=jax.ShapeDtypeStruct((q, 1, WAY), jnp.float32),
        compiler_params=pltpu.CompilerParams(
            dimension_semantics=("arbitrary",)),
    )(nq, msb, agg, thr)
    return out.reshape(q, WAY)


# speedup vs baseline: 1.0262x; 1.0262x over previous
import jax
import jax.numpy as jnp
from jax.experimental import pallas as pl
from jax.experimental.pallas import tpu as pltpu

WAY = 5
K = 3
NEG = -1e9
EPS = 1e-30
CHUNK = 64


def _support_prep_kernel(s_ref, ms_ref, msb_ref):
    s = s_ref[...]
    nrm = jnp.sqrt(jnp.sum(s * s, axis=1, keepdims=True))
    ns = s / jnp.maximum(nrm, EPS)
    hw = s.shape[2]
    rows = []
    for i in range(WAY):
        for j in range(WAY):
            rows.append(jnp.sum(ns[i] * ns[j], axis=0, keepdims=True))
    sim = jnp.concatenate(rows, axis=0)
    sim = sim - jnp.max(sim, axis=1, keepdims=True)
    e = jnp.exp(sim)
    w = e / jnp.sum(e, axis=1, keepdims=True)
    aggs = [jnp.mean(w[i * WAY:(i + 1) * WAY], axis=0, keepdims=True)
            for i in range(WAY)]
    agg = jnp.concatenate(aggs, axis=0)
    n = WAY * hw
    mean = jnp.sum(agg) / n
    var = jnp.sum((agg - mean) ** 2) / (n - 1)
    thr = mean - jnp.sqrt(var)
    mask = (agg > thr).astype(jnp.float32)
    ms = ns * mask[:, None, :]
    ms_ref[...] = ms
    msb_ref[...] = ms.astype(jnp.bfloat16)


def _query_agg_kernel(x_ref, ms_ref, agg_ref, nq_ref):
    qb, c, hw = x_ref.shape
    for qi in range(qb):
        acc = None
        for cc in range(0, c, CHUNK):
            xc = x_ref[qi, cc:cc + CHUNK, :]
            p = jnp.sum(xc * xc, axis=0, keepdims=True)
            acc = p if acc is None else acc + p
        inv = 1.0 / jnp.maximum(jnp.sqrt(acc), EPS)
        sims = [None] * WAY
        for cc in range(0, c, CHUNK):
            xc = x_ref[qi, cc:cc + CHUNK, :] * inv
            nq_ref[qi, cc:cc + CHUNK, :] = xc.astype(jnp.bfloat16)
            for w in range(WAY):
                p = jnp.sum(xc * ms_ref[w, cc:cc + CHUNK, :],
                            axis=0, keepdims=True)
                sims[w] = p if sims[w] is None else sims[w] + p
        sim = jnp.concatenate(sims, axis=0)
        sim = sim - jnp.max(sim, axis=1, keepdims=True)
        e = jnp.exp(sim)
        w_ = e / jnp.sum(e, axis=1, keepdims=True)
        agg_ref[qi] = jnp.mean(w_, axis=0, keepdims=True)


def _thr_kernel(agg_ref, thr_ref):
    a = agg_ref[...]
    n = a.shape[0] * a.shape[2]
    mean = jnp.sum(a) / n
    var = jnp.sum((a - mean) ** 2) / (n - 1)
    thr_ref[...] = jnp.broadcast_to(mean - jnp.sqrt(var), (1, 1))


def _top3_rowsum(inner):
    m1 = jnp.max(inner, axis=1, keepdims=True)
    ge1 = inner >= m1
    c1 = jnp.sum(ge1.astype(jnp.float32), axis=1, keepdims=True)
    x2 = jnp.where(ge1, NEG, inner)
    m2 = jnp.max(x2, axis=1, keepdims=True)
    ge2 = x2 >= m2
    c2 = jnp.sum(ge2.astype(jnp.float32), axis=1, keepdims=True)
    x3 = jnp.where(ge2, NEG, x2)
    m3 = jnp.max(x3, axis=1, keepdims=True)
    kf = float(K)
    t1 = jnp.minimum(c1, kf)
    t2 = jnp.minimum(c2, jnp.maximum(kf - t1, 0.0))
    t3 = jnp.maximum(kf - t1 - t2, 0.0)
    return m1 * t1 + m2 * t2 + m3 * t3


def _main_kernel(nq_ref, msb_ref, agg_ref, thr_ref, out_ref):
    thr = thr_ref[0, 0]
    mask = (agg_ref[0] > thr).astype(jnp.bfloat16)
    mq = nq_ref[0] * mask
    sums = []
    for w in range(WAY):
        inner = jax.lax.dot_general(
            mq, msb_ref[w], (((0,), (0,)), ((), ())),
            preferred_element_type=jnp.float32)
        sums.append(jnp.sum(_top3_rowsum(inner), axis=0, keepdims=True))
    out_ref[...] = jnp.concatenate(sums, axis=1)[None]


def kernel(x1, x2):
    q, c, h, w = x1.shape[0], x1.shape[1], x1.shape[2], x1.shape[3]
    hw = h * w
    xq = x1.reshape(q, c, hw)
    xs = x2.reshape(WAY, c, hw)

    ms, msb = pl.pallas_call(
        _support_prep_kernel,
        out_shape=(jax.ShapeDtypeStruct((WAY, c, hw), jnp.float32),
                   jax.ShapeDtypeStruct((WAY, c, hw), jnp.bfloat16)),
    )(xs)

    qb = 8
    agg, nq = pl.pallas_call(
        _query_agg_kernel,
        grid=(q // qb,),
        in_specs=[
            pl.BlockSpec((qb, c, hw), lambda i: (i, 0, 0)),
            pl.BlockSpec((WAY, c, hw), lambda i: (0, 0, 0)),
        ],
        out_specs=(pl.BlockSpec((qb, 1, hw), lambda i: (i, 0, 0)),
                   pl.BlockSpec((qb, c, hw), lambda i: (i, 0, 0))),
        out_shape=(jax.ShapeDtypeStruct((q, 1, hw), jnp.float32),
                   jax.ShapeDtypeStruct((q, c, hw), jnp.bfloat16)),
        compiler_params=pltpu.CompilerParams(
            dimension_semantics=("arbitrary",)),
    )(xq, ms)

    thr = pl.pallas_call(
        _thr_kernel,
        out_shape=jax.ShapeDtypeStruct((1, 1), jnp.float32),
    )(agg)

    out = pl.pallas_call(
        _main_kernel,
        grid=(q,),
        in_specs=[
            pl.BlockSpec((1, c, hw), lambda i: (i, 0, 0)),
            pl.BlockSpec((WAY, c, hw), lambda i: (0, 0, 0)),
            pl.BlockSpec((1, 1, hw), lambda i: (i, 0, 0)),
            pl.BlockSpec((1, 1), lambda i: (0, 0)),
        ],
        out_specs=pl.BlockSpec((1, 1, WAY), lambda i: (i, 0, 0)),
        out_shape=jax.ShapeDtypeStruct((q, 1, WAY), jnp.float32),
        compiler_params=pltpu.CompilerParams(
            dimension_semantics=("arbitrary",)),
    )(nq, msb, agg, thr)
    return out.reshape(q, WAY)
